# baseline (device time: 127854 ns/iter reference)
import jax
import jax.numpy as jnp
from jax import lax
from jax.experimental import pallas as pl
from jax.experimental.pallas import tpu as pltpu

N_DEV = 8
B = 2
SQ_LOC = 128
D_MODEL = 512
HQ = 32
DH = 64
D_FF = HQ * DH
CHUNK = D_FF // N_DEV
SKV = 128


def _f(t):
    return jnp.where(t < 4, t, 11 - t)


def kernel(x, Wq, K_ext, V_ext, Wo):
    def body(x_ref, wq_ref, k_ref, v_ref, wo_ref, out_ref,
             wq_full, wo_full, ctx_buf, send_sems, recv_sems):
        me = lax.axis_index("i")
        my_cpos = _f(me)
        nxt = _f(jnp.mod(my_cpos + 1, N_DEV))
        prv = _f(jnp.mod(my_cpos - 1, N_DEV))

        barrier = pltpu.get_barrier_semaphore()
        for nbr in (nxt, prv):
            pl.semaphore_signal(
                barrier, inc=1, device_id=(nbr,),
                device_id_type=pl.DeviceIdType.MESH,
            )
        pl.semaphore_wait(barrier, 2)

        wq_full[:, pl.ds(me * CHUNK, CHUNK)] = wq_ref[...]
        wo_full[pl.ds(me * CHUNK, CHUNK), :] = wo_ref[...]

        for h in range(N_DEV - 1):
            o = _f(jnp.mod(my_cpos - h, N_DEV))
            rq = pltpu.make_async_remote_copy(
                src_ref=wq_full.at[:, pl.ds(o * CHUNK, CHUNK)],
                dst_ref=wq_full.at[:, pl.ds(o * CHUNK, CHUNK)],
                send_sem=send_sems.at[0, h],
                recv_sem=recv_sems.at[0, h],
                device_id=(nxt,),
                device_id_type=pl.DeviceIdType.MESH,
            )
            ro = pltpu.make_async_remote_copy(
                src_ref=wo_full.at[pl.ds(o * CHUNK, CHUNK), :],
                dst_ref=wo_full.at[pl.ds(o * CHUNK, CHUNK), :],
                send_sem=send_sems.at[1, h],
                recv_sem=recv_sems.at[1, h],
                device_id=(nxt,),
                device_id_type=pl.DeviceIdType.MESH,
            )
            rq.start()
            ro.start()
            rq.wait()
            ro.wait()

        x2d = x_ref[...].reshape(B * SQ_LOC, D_MODEL)
        q2d = jnp.dot(x2d, wq_full[...], preferred_element_type=jnp.float32)
        q4 = q2d.reshape(B, SQ_LOC, HQ, DH)

        ii = lax.broadcasted_iota(jnp.int32, (SQ_LOC, SKV), 0)
        jj = lax.broadcasted_iota(jnp.int32, (SQ_LOC, SKV), 1)
        qb = 2 * me + ii // 64
        kb = jj // 64
        mask = (qb == kb) | (jnp.mod(qb, 4) == jnp.mod(kb, 4))
        row_keep = jnp.mod(qb, 4) < 2

        for b in range(B):
            for hh in range(HQ):
                qbh = q4[b, :, hh, :]
                kbh = k_ref[b, :, hh, :]
                s = lax.dot_general(
                    qbh, kbh,
                    dimension_numbers=(((1,), (1,)), ((), ())),
                    preferred_element_type=jnp.float32,
                ) * 0.125
                s = jnp.where(mask, s, -1e9)
                m = jnp.max(s, axis=1, keepdims=True)
                w = jnp.exp(s - m)
                wsum = jnp.sum(w, axis=1, keepdims=True)
                w = jnp.where(row_keep, w / wsum, 0.0)
                ctx = jnp.dot(w, v_ref[b, :, hh, :],
                              preferred_element_type=jnp.float32)
                ctx_buf[pl.ds(b * SQ_LOC, SQ_LOC), pl.ds(hh * DH, DH)] = ctx

        out2d = jnp.dot(ctx_buf[...], wo_full[...],
                        preferred_element_type=jnp.float32)
        out_ref[...] = out2d.reshape(B, SQ_LOC, D_MODEL)

    return pl.pallas_call(
        body,
        out_shape=jax.ShapeDtypeStruct((B, SQ_LOC, D_MODEL), jnp.float32),
        in_specs=[
            pl.BlockSpec(memory_space=pltpu.VMEM),
            pl.BlockSpec(memory_space=pltpu.VMEM),
            pl.BlockSpec(memory_space=pltpu.VMEM),
            pl.BlockSpec(memory_space=pltpu.VMEM),
            pl.BlockSpec(memory_space=pltpu.VMEM),
        ],
        out_specs=pl.BlockSpec(memory_space=pltpu.VMEM),
        scratch_shapes=[
            pltpu.VMEM((D_MODEL, D_FF), jnp.float32),
            pltpu.VMEM((D_FF, D_MODEL), jnp.float32),
            pltpu.VMEM((B * SQ_LOC, D_FF), jnp.float32),
            pltpu.SemaphoreType.DMA((2, N_DEV - 1)),
            pltpu.SemaphoreType.DMA((2, N_DEV - 1)),
        ],
        compiler_params=pltpu.CompilerParams(collective_id=0),
    )(x, Wq, K_ext, V_ext, Wo)


# device time: 51419 ns/iter; 2.4865x vs baseline; 2.4865x over previous
import jax
import jax.numpy as jnp
from jax import lax
from jax.experimental import pallas as pl
from jax.experimental.pallas import tpu as pltpu

N_DEV = 8
B = 2
SQ_LOC = 128
D_MODEL = 512
HQ = 32
DH = 64
D_FF = HQ * DH
CHUNK = D_FF // N_DEV
SKV = 128
CW_HOPS = 4
CCW_HOPS = 3


def _f(t):
    return jnp.where(t < 4, t, 11 - t)


def kernel(x, Wq, K_ext, V_ext, Wo):
    def body(x_ref, wq_ref, k_ref, v_ref, wo_ref, out_ref,
             wq_full, wo_full, xb, k_hm, v_hm, q_hm, ctx_buf,
             cw_send, cw_recv, ccw_send, ccw_recv):
        me = lax.axis_index("i")
        cp = _f(me)
        nxt = _f(jnp.mod(cp + 1, N_DEV))
        prv = _f(jnp.mod(cp - 1, N_DEV))
        is_even = jnp.mod(me, 2) == 0

        wq_full[:, pl.ds(me * CHUNK, CHUNK)] = wq_ref[...].astype(jnp.bfloat16)
        wo_full[pl.ds(me * CHUNK, CHUNK), :] = wo_ref[...].astype(jnp.bfloat16)

        barrier = pltpu.get_barrier_semaphore()
        for nbr in (nxt, prv):
            pl.semaphore_signal(
                barrier, inc=1, device_id=(nbr,),
                device_id_type=pl.DeviceIdType.MESH,
            )
        pl.semaphore_wait(barrier, 2)

        def start_pair(slot, sems_s, sems_r, h, target):
            rq = pltpu.make_async_remote_copy(
                src_ref=wq_full.at[:, pl.ds(slot * CHUNK, CHUNK)],
                dst_ref=wq_full.at[:, pl.ds(slot * CHUNK, CHUNK)],
                send_sem=sems_s.at[0, h],
                recv_sem=sems_r.at[0, h],
                device_id=(target,),
                device_id_type=pl.DeviceIdType.MESH,
            )
            ro = pltpu.make_async_remote_copy(
                src_ref=wo_full.at[pl.ds(slot * CHUNK, CHUNK), :],
                dst_ref=wo_full.at[pl.ds(slot * CHUNK, CHUNK), :],
                send_sem=sems_s.at[1, h],
                recv_sem=sems_r.at[1, h],
                device_id=(target,),
                device_id_type=pl.DeviceIdType.MESH,
            )
            rq.start()
            ro.start()
            return rq, ro

        def start_cw(h):
            return start_pair(_f(jnp.mod(cp - h, N_DEV)), cw_send, cw_recv,
                              h, nxt)

        def start_ccw(h):
            return start_pair(_f(jnp.mod(cp + h, N_DEV)), ccw_send, ccw_recv,
                              h, prv)

        cw = start_cw(0)
        ccw = start_ccw(0)

        @pl.when(is_even)
        def _():
            xb[...] = x_ref[...].reshape(B * SQ_LOC, D_MODEL).astype(
                jnp.bfloat16)
            for b in range(B):
                for hh in range(HQ):
                    k_hm[b * HQ + hh] = k_ref[b, :, hh, :].astype(jnp.bfloat16)
                    v_hm[b * HQ + hh] = v_ref[b, :, hh, :].astype(jnp.bfloat16)

        for r in cw + ccw:
            r.wait()

        for h in range(1, CW_HOPS):
            cw = start_cw(h)
            ccw = start_ccw(h) if h < CCW_HOPS else None
            for r in cw:
                r.wait()
            if ccw is not None:
                for r in ccw:
                    r.wait()

        @pl.when(is_even)
        def _():
            q2d = jnp.dot(xb[...], wq_full[...],
                          preferred_element_type=jnp.float32)
            q4 = q2d.reshape(B, SQ_LOC, HQ, DH).astype(jnp.bfloat16)
            for b in range(B):
                for hh in range(HQ):
                    q_hm[b * HQ + hh] = q4[b, :, hh, :]

            qv = q_hm[...].reshape(B * HQ * 2, 64, DH)
            kv = k_hm[...].reshape(B * HQ * 2, 64, DH)
            vv = v_hm[...].reshape(B * HQ * 2, 64, DH)
            s = lax.dot_general(
                qv, kv,
                dimension_numbers=(((2,), (2,)), ((0,), (0,))),
                preferred_element_type=jnp.float32,
            ) * 0.125
            m = jnp.max(s, axis=-1, keepdims=True)
            w = jnp.exp(s - m)
            wsum = jnp.sum(w, axis=-1, keepdims=True)
            w = (w / wsum).astype(jnp.bfloat16)
            ctx = lax.dot_general(
                w, vv,
                dimension_numbers=(((2,), (1,)), ((0,), (0,))),
                preferred_element_type=jnp.float32,
            ).reshape(B * HQ, SQ_LOC, DH)
            for b in range(B):
                for hh in range(HQ):
                    ctx_buf[pl.ds(b * SQ_LOC, SQ_LOC), pl.ds(hh * DH, DH)] = (
                        ctx[b * HQ + hh].astype(jnp.bfloat16))

            out2d = jnp.dot(ctx_buf[...], wo_full[...],
                            preferred_element_type=jnp.float32)
            out_ref[...] = out2d.reshape(B, SQ_LOC, D_MODEL)

        @pl.when(jnp.logical_not(is_even))
        def _():
            out_ref[...] = jnp.zeros((B, SQ_LOC, D_MODEL), jnp.float32)

    return pl.pallas_call(
        body,
        out_shape=jax.ShapeDtypeStruct((B, SQ_LOC, D_MODEL), jnp.float32),
        in_specs=[pl.BlockSpec(memory_space=pltpu.VMEM)] * 5,
        out_specs=pl.BlockSpec(memory_space=pltpu.VMEM),
        scratch_shapes=[
            pltpu.VMEM((D_MODEL, D_FF), jnp.bfloat16),
            pltpu.VMEM((D_FF, D_MODEL), jnp.bfloat16),
            pltpu.VMEM((B * SQ_LOC, D_MODEL), jnp.bfloat16),
            pltpu.VMEM((B * HQ, SKV, DH), jnp.bfloat16),
            pltpu.VMEM((B * HQ, SKV, DH), jnp.bfloat16),
            pltpu.VMEM((B * HQ, SQ_LOC, DH), jnp.bfloat16),
            pltpu.VMEM((B * SQ_LOC, D_FF), jnp.bfloat16),
            pltpu.SemaphoreType.DMA((2, CW_HOPS)),
            pltpu.SemaphoreType.DMA((2, CW_HOPS)),
            pltpu.SemaphoreType.DMA((2, CCW_HOPS)),
            pltpu.SemaphoreType.DMA((2, CCW_HOPS)),
        ],
        compiler_params=pltpu.CompilerParams(collective_id=0),
    )(x, Wq, K_ext, V_ext, Wo)
